# Initial kernel scaffold; baseline (speedup 1.0000x reference)
import jax
import jax.numpy as jnp
from jax.experimental import pallas as pl


def kernel(x, edge_index, edge_attr, lin1_W, lin1_b, root1, bias1, bn1_g, bn1_b, lin2_W, lin2_b, root2, bias2, bn2_g, bn2_b, lin3_W, lin3_b, root3, bias3, bn3_g, bn3_b, me1_W, me1_b, me2_W, me2_b, me3_W, me3_b, out_W, out_b):
    raise NotImplementedError("write your pallas kernel here")


# speedup vs baseline: 11.2262x; 11.2262x over previous
import functools

import jax
import jax.numpy as jnp
from jax.experimental import pallas as pl

N = 35
E = N * N

_BN_INV = 1.0 / (1.0 + 0.001) ** 0.5


def _dot(a, b):
    return jnp.dot(a, b, preferred_element_type=jnp.float32)


def _dotT(a, b):
    return jax.lax.dot_general(a, b, (((0,), (0,)), ((), ())),
                               preferred_element_type=jnp.float32)


def _fused_kernel(x_ref, ei_ref, ea_ref,
                  lin1W_ref, lin1b_ref, root1_ref, bias1_ref, bn1g_ref, bn1b_ref,
                  lin2W_ref, lin2b_ref, root2_ref, bias2_ref, bn2g_ref, bn2b_ref,
                  lin3W_ref, lin3b_ref, root3_ref, bias3_ref, bn3g_ref, bn3b_ref,
                  me1W_ref, me1b_ref, me2W_ref, me2b_ref, me3W_ref, me3b_ref,
                  outW1_ref, outW2_ref, outb_ref,
                  out_ref):
    x = x_ref[...]
    ea = ea_ref[...]

    node_iota = jax.lax.broadcasted_iota(jnp.int32, (N, E), 0)
    src_row = ei_ref[0:1, :]
    dst_row = ei_ref[1:2, :]
    ST = (node_iota == src_row).astype(jnp.float32)
    D = (node_iota == dst_row).astype(jnp.float32)

    cnt = jnp.maximum(jnp.sum(D, axis=1, keepdims=True), 1.0)
    inv_cnt = 1.0 / cnt

    row_iota = jax.lax.broadcasted_iota(jnp.int32, (N, N), 0)
    col_iota = jax.lax.broadcasted_iota(jnp.int32, (N, N), 1)
    off_diag = (row_iota != col_iota).astype(jnp.float32)

    def transpose(m):
        eye = (row_iota == col_iota).astype(jnp.float32)
        return jax.lax.dot_general(eye, m, (((1,), (1,)), ((), ())),
                                   preferred_element_type=jnp.float32)

    w1 = jax.nn.relu(_dot(ea, lin1W_ref[...]) + lin1b_ref[...])
    xs1 = _dotT(ST, x)
    s1 = _dot(D, xs1 * w1)
    h1 = s1 * inv_cnt + _dot(x, root1_ref[...]) + bias1_ref[...]
    x1 = jax.nn.sigmoid(h1 * _BN_INV * bn1g_ref[...] + bn1b_ref[...])
    x1 = (x1 + transpose(x1)) * 0.5 * off_diag

    w2 = jax.nn.relu(_dot(ea, lin2W_ref[...]) + lin2b_ref[...])
    xs2 = _dotT(ST, x1)
    msg2 = jnp.sum(xs2 * w2, axis=1, keepdims=True)
    s2 = _dot(D, msg2)
    h2 = s2 * inv_cnt + _dot(x1, root2_ref[...]) + bias2_ref[...]
    x2 = jax.nn.sigmoid(h2 * _BN_INV * bn2g_ref[...] + bn2b_ref[...])

    w3 = jax.nn.relu(_dot(ea, lin3W_ref[...]) + lin3b_ref[...])
    xs3 = _dotT(ST, x2)
    s3 = _dot(D, xs3 * w3)
    h3 = s3 * inv_cnt + _dot(x2, root3_ref[...]) + bias3_ref[...]
    x3 = jax.nn.sigmoid(h3 * _BN_INV * bn3g_ref[...] + bn3b_ref[...])

    x6 = (x3 + x1) * 0.5
    x6 = (x6 + transpose(x6)) * 0.5 * off_diag

    ef = jax.nn.relu(_dot(ea, me1W_ref[...]) + me1b_ref[...])
    ef = jax.nn.relu(_dot(ef, me2W_ref[...]) + me2b_ref[...])
    ef = _dot(ef, me3W_ref[...]) + me3b_ref[...]

    e_iota = jax.lax.broadcasted_iota(jnp.int32, (N, E), 1)
    P = (e_iota // N == node_iota).astype(jnp.float32)
    T = (e_iota % N == node_iota).astype(jnp.float32)
    W2e = _dotT(T, outW2_ref[...])
    edge_term = _dot(P, ef * W2e)
    out_ref[...] = (_dot(x6, outW1_ref[...]) + edge_term + outb_ref[...])


@functools.partial(jax.jit, static_argnames=("interpret",))
def _run(x, edge_index, edge_attr, args, interpret=False):
    return pl.pallas_call(
        _fused_kernel,
        out_shape=jax.ShapeDtypeStruct((N, N), jnp.float32),
        interpret=interpret,
    )(x, edge_index, edge_attr, *args)


def kernel(x, edge_index, edge_attr, lin1_W, lin1_b, root1, bias1, bn1_g,
           bn1_b, lin2_W, lin2_b, root2, bias2, bn2_g, bn2_b, lin3_W, lin3_b,
           root3, bias3, bn3_g, bn3_b, me1_W, me1_b, me2_W, me2_b, me3_W,
           me3_b, out_W, out_b):
    x = x.astype(jnp.float32)
    ea = edge_attr.astype(jnp.float32)
    args = (
        lin1_W, lin1_b.reshape(1, N), root1, bias1.reshape(1, N),
        bn1_g.reshape(1, N), bn1_b.reshape(1, N),
        lin2_W, lin2_b.reshape(1, N), root2, bias2.reshape(1, 1),
        bn2_g.reshape(1, 1), bn2_b.reshape(1, 1),
        lin3_W, lin3_b.reshape(1, N), root3, bias3.reshape(1, N),
        bn3_g.reshape(1, N), bn3_b.reshape(1, N),
        me1_W, me1_b.reshape(1, 128), me2_W, me2_b.reshape(1, 256),
        me3_W, me3_b.reshape(1, 1),
        out_W[:N], out_W[N:], out_b.reshape(1, N),
    )
    return _run(x, edge_index, ea, args)
